# Initial kernel scaffold; baseline (speedup 1.0000x reference)
import jax
import jax.numpy as jnp
from jax.experimental import pallas as pl


def kernel(data, segment_ids):
    raise NotImplementedError("write your pallas kernel here")


# speedup vs baseline: 8.0994x; 8.0994x over previous
import functools

import jax
import jax.numpy as jnp
from jax import lax
from jax.experimental import pallas as pl
from jax.experimental.pallas import tpu as pltpu
from jax.experimental.pallas import tpu_sc as plsc

NSEG = 10000
D = 128
N = 320000
NC = 2
NS = 16
NW = NC * NS
ROWS_PER_TILE = N // NW
CHUNK = 125
NCHUNK = ROWS_PER_TILE // CHUNK
SEG_PER_TILE = NSEG // NS
CW = 16

_mesh = plsc.VectorSubcoreMesh(core_axis_name="c", subcore_axis_name="s")


@functools.partial(
    pl.kernel,
    mesh=_mesh,
    compiler_params=pltpu.CompilerParams(use_tc_tiling_on_sc=False),
    out_type=[
        jax.ShapeDtypeStruct((NC * NSEG, D), jnp.float32),
        jax.ShapeDtypeStruct((NC * NSEG, CW), jnp.float32),
    ],
    scratch_types=[
        pltpu.VMEM((NCHUNK, CHUNK), jnp.int32),
        pltpu.VMEM((CHUNK, D), jnp.float32),
        pltpu.VMEM((CHUNK, CW), jnp.float32),
        pltpu.VMEM((SEG_PER_TILE, CW), jnp.float32),
        pltpu.VMEM_SHARED((NSEG, D), jnp.float32),
        pltpu.VMEM_SHARED((NSEG, CW), jnp.float32),
    ],
)
def _scatter_stage(data_hbm, ids_hbm, zrows_hbm, ones_hbm, zcnt_hbm,
                   psums_hbm, pcnts_hbm,
                   ids_v, rows_v, ones_v, cnt_v, ssum, scnt):
    cid = lax.axis_index("c")
    sid = lax.axis_index("s")
    wid = sid * NC + cid
    row0 = wid * ROWS_PER_TILE
    seg0 = sid * SEG_PER_TILE

    pltpu.sync_copy(ids_hbm.at[pl.ds(wid * NCHUNK, NCHUNK)], ids_v)
    pltpu.sync_copy(ones_hbm, ones_v)
    pltpu.sync_copy(zcnt_hbm, cnt_v)
    pltpu.sync_copy(zrows_hbm, rows_v)

    for k in range(SEG_PER_TILE // CHUNK):
        pltpu.sync_copy(rows_v, ssum.at[pl.ds(seg0 + k * CHUNK, CHUNK)])
    pltpu.sync_copy(cnt_v, scnt.at[pl.ds(seg0, SEG_PER_TILE)])
    plsc.subcore_barrier()

    def step(j, carry):
        pltpu.sync_copy(data_hbm.at[pl.ds(row0 + j * CHUNK, CHUNK)], rows_v)
        pltpu.sync_copy(rows_v, ssum.at[ids_v.at[j]], add=True)
        pltpu.sync_copy(ones_v, scnt.at[ids_v.at[j]], add=True)
        return carry

    lax.fori_loop(0, NCHUNK, step, 0)
    plsc.subcore_barrier()

    out0 = cid * NSEG + seg0
    for k in range(SEG_PER_TILE // CHUNK):
        pltpu.sync_copy(ssum.at[pl.ds(seg0 + k * CHUNK, CHUNK)], rows_v)
        pltpu.sync_copy(rows_v, psums_hbm.at[pl.ds(out0 + k * CHUNK, CHUNK)])
    pltpu.sync_copy(scnt.at[pl.ds(seg0, SEG_PER_TILE)], cnt_v)
    pltpu.sync_copy(cnt_v, pcnts_hbm.at[pl.ds(out0, SEG_PER_TILE)])


_FR = 1000


def _fin_body(s_ref, c_ref, o_ref):
    s = s_ref[0] + s_ref[1]
    c = c_ref[0] + c_ref[1]
    cnt = jnp.sum(c, axis=1, keepdims=True)
    o_ref[...] = s / jnp.maximum(cnt, 1.0)


_finalize = pl.pallas_call(
    _fin_body,
    grid=(NSEG // _FR,),
    in_specs=[
        pl.BlockSpec((NC, _FR, D), lambda g: (0, g, 0)),
        pl.BlockSpec((NC, _FR, CW), lambda g: (0, g, 0)),
    ],
    out_specs=pl.BlockSpec((_FR, D), lambda g: (g, 0)),
    out_shape=jax.ShapeDtypeStruct((NSEG, D), jnp.float32),
)


@jax.jit
def kernel(data, segment_ids):
    ids = segment_ids.astype(jnp.int32).reshape(NW * NCHUNK, CHUNK)
    zrows = jnp.zeros((CHUNK, D), jnp.float32)
    ones = jnp.full((CHUNK, CW), 1.0 / CW, jnp.float32)
    zcnt = jnp.zeros((SEG_PER_TILE, CW), jnp.float32)
    psums, pcnts = _scatter_stage(data, ids, zrows, ones, zcnt)
    return _finalize(psums.reshape(NC, NSEG, D), pcnts.reshape(NC, NSEG, CW))


# speedup vs baseline: 12.0088x; 1.4827x over previous
import functools

import jax
import jax.numpy as jnp
from jax import lax
from jax.experimental import pallas as pl
from jax.experimental.pallas import tpu as pltpu
from jax.experimental.pallas import tpu_sc as plsc

NSEG = 10000
D = 128
N = 320000
NC = 2
NS = 16
NW = NC * NS
ROWS_PER_TILE = N // NW
CHUNK = 125
NCHUNK = ROWS_PER_TILE // CHUNK
SEG_PER_TILE = NSEG // NS
CW = 16

_mesh = plsc.VectorSubcoreMesh(core_axis_name="c", subcore_axis_name="s")


@functools.partial(
    pl.kernel,
    mesh=_mesh,
    compiler_params=pltpu.CompilerParams(use_tc_tiling_on_sc=False),
    out_type=[
        jax.ShapeDtypeStruct((NC * NSEG, D), jnp.float32),
        jax.ShapeDtypeStruct((NC * NSEG, CW), jnp.float32),
    ],
    scratch_types=[
        pltpu.VMEM((1, CHUNK), jnp.int32),
        pltpu.VMEM((1, CHUNK), jnp.int32),
        pltpu.VMEM((CHUNK, D), jnp.float32),
        pltpu.VMEM((CHUNK, D), jnp.float32),
        pltpu.VMEM((CHUNK, CW), jnp.float32),
        pltpu.VMEM_SHARED((NSEG, D), jnp.float32),
        pltpu.VMEM_SHARED((NSEG, CW), jnp.float32),
        pltpu.SemaphoreType.DMA,
        pltpu.SemaphoreType.DMA,
        pltpu.SemaphoreType.DMA,
        pltpu.SemaphoreType.DMA,
    ],
)
def _scatter_stage(data_hbm, ids_hbm, zrows_hbm, ones_hbm, zcnt_hbm,
                   psums_hbm, pcnts_hbm,
                   ids_a, ids_b, rows_a, rows_b, ones_v, ssum, scnt,
                   sem_a, sem_b, sem_ia, sem_ib):
    cid = lax.axis_index("c")
    sid = lax.axis_index("s")
    wid = sid * NC + cid
    row0 = wid * ROWS_PER_TILE
    id0 = wid * NCHUNK
    seg0 = sid * SEG_PER_TILE

    pltpu.sync_copy(zrows_hbm, rows_a)
    for k in range(SEG_PER_TILE // CHUNK):
        pltpu.sync_copy(rows_a, ssum.at[pl.ds(seg0 + k * CHUNK, CHUNK)])
    pltpu.sync_copy(zcnt_hbm, ones_v)
    for k in range(SEG_PER_TILE // CHUNK):
        pltpu.sync_copy(ones_v, scnt.at[pl.ds(seg0 + k * CHUNK, CHUNK)])
    pltpu.sync_copy(ones_hbm, ones_v)
    plsc.subcore_barrier()

    pltpu.async_copy(data_hbm.at[pl.ds(row0, CHUNK)], rows_a, sem_a)
    pltpu.async_copy(ids_hbm.at[pl.ds(id0, 1)], ids_a, sem_ia)
    pltpu.async_copy(data_hbm.at[pl.ds(row0 + CHUNK, CHUNK)], rows_b, sem_b)
    pltpu.async_copy(ids_hbm.at[pl.ds(id0 + 1, 1)], ids_b, sem_ib)

    def step(i, carry):
        bufs = ((rows_a, ids_a, sem_a, sem_ia), (rows_b, ids_b, sem_b, sem_ib))
        for b, (buf, idb, sem, isem) in enumerate(bufs):
            j = i * 2 + b
            pltpu.make_async_copy(data_hbm.at[pl.ds(row0, CHUNK)], buf, sem).wait()
            pltpu.make_async_copy(ids_hbm.at[pl.ds(id0, 1)], idb, isem).wait()
            pltpu.sync_copy(buf, ssum.at[idb.at[0]], add=True)
            pltpu.sync_copy(ones_v, scnt.at[idb.at[0]], add=True)

            @pl.when(j + 2 < NCHUNK)
            def _():
                pltpu.async_copy(
                    data_hbm.at[pl.ds(row0 + (j + 2) * CHUNK, CHUNK)], buf, sem)
                pltpu.async_copy(ids_hbm.at[pl.ds(id0 + j + 2, 1)], idb, isem)
        return carry

    lax.fori_loop(0, NCHUNK // 2, step, 0)
    plsc.subcore_barrier()

    out0 = cid * NSEG + seg0
    for k in range(SEG_PER_TILE // CHUNK):
        pltpu.sync_copy(ssum.at[pl.ds(seg0 + k * CHUNK, CHUNK)], rows_a)
        pltpu.sync_copy(rows_a, psums_hbm.at[pl.ds(out0 + k * CHUNK, CHUNK)])
    for k in range(SEG_PER_TILE // CHUNK):
        pltpu.sync_copy(scnt.at[pl.ds(seg0 + k * CHUNK, CHUNK)], ones_v)
        pltpu.sync_copy(ones_v, pcnts_hbm.at[pl.ds(out0 + k * CHUNK, CHUNK)])


_FR = 1000


def _fin_body(s_ref, c_ref, o_ref):
    s = s_ref[0] + s_ref[1]
    c = c_ref[0] + c_ref[1]
    cnt = jnp.sum(c, axis=1, keepdims=True)
    o_ref[...] = s / jnp.maximum(cnt, 1.0)


_finalize = pl.pallas_call(
    _fin_body,
    grid=(NSEG // _FR,),
    in_specs=[
        pl.BlockSpec((NC, _FR, D), lambda g: (0, g, 0)),
        pl.BlockSpec((NC, _FR, CW), lambda g: (0, g, 0)),
    ],
    out_specs=pl.BlockSpec((_FR, D), lambda g: (g, 0)),
    out_shape=jax.ShapeDtypeStruct((NSEG, D), jnp.float32),
)


@jax.jit
def kernel(data, segment_ids):
    ids = segment_ids.astype(jnp.int32).reshape(NW * NCHUNK, CHUNK)
    zrows = jnp.zeros((CHUNK, D), jnp.float32)
    ones = jnp.full((CHUNK, CW), 1.0 / CW, jnp.float32)
    zcnt = jnp.zeros((CHUNK, CW), jnp.float32)
    psums, pcnts = _scatter_stage(data, ids, zrows, ones, zcnt)
    return _finalize(psums.reshape(NC, NSEG, D), pcnts.reshape(NC, NSEG, CW))
